# Initial kernel scaffold; baseline (speedup 1.0000x reference)
import jax
import jax.numpy as jnp
from jax.experimental import pallas as pl


def kernel(x, edge_index, edge_weight, W, b):
    raise NotImplementedError("write your pallas kernel here")


# speedup vs baseline: 4.3930x; 4.3930x over previous
import functools

import jax
import jax.numpy as jnp
from jax import lax
from jax.experimental import pallas as pl
from jax.experimental.pallas import tpu as pltpu
from jax.experimental.pallas import tpu_sc as plsc

NC = 2
NS = 16
L = 16
NW = NC * NS


def _matmul_body(x_ref, wt_ref, b_ref, o_ref):
    o_ref[...] = (
        jnp.dot(x_ref[...], wt_ref[...], preferred_element_type=jnp.float32)
        + b_ref[...]
    )


def _fuse_body(p0_ref, p1_ref, x_ref, o_ref):
    y = p0_ref[...] + p1_ref[...] + x_ref[...]
    o_ref[...] = jnp.where(y >= 0, y, 0.2 * y)


def _edge_body(n, cpw, h_hbm, cols_hbm, rows_hbm, w_hbm, p_hbm,
               cols_v, rows_v, w_v, msgs_v, acc):
    C = 128
    D = 128
    cid = lax.axis_index("c")
    sid = lax.axis_index("s")
    wid = sid * NC + cid

    pltpu.sync_copy(cols_hbm.at[wid], cols_v)
    pltpu.sync_copy(rows_hbm.at[wid], rows_v)
    pltpu.sync_copy(w_hbm.at[wid], w_v)

    zeros16 = jnp.zeros((L,), jnp.float32)

    def zrow(r, carry):
        for d in range(D // L):
            msgs_v[r, pl.ds(d * L, L)] = zeros16
        return carry

    lax.fori_loop(0, C, zrow, 0)
    rpt = (n // (NS * 8)) * 8
    rem = n - NS * rpt

    def zero_acc_rows(base, count):
        full, tail = count // C, count % C
        for k in range(full):
            pltpu.sync_copy(msgs_v,
                            acc.at[pl.ds(pl.multiple_of(base + k * C, 8), C)])
        if tail:
            pltpu.sync_copy(msgs_v.at[pl.ds(0, tail)],
                            acc.at[pl.ds(pl.multiple_of(base + full * C, 8), tail)])

    zero_acc_rows(sid * rpt, rpt)
    if rem:
        @pl.when(sid == NS - 1)
        def _():
            zero_acc_rows(NS * rpt, rem)
    plsc.subcore_barrier()

    def chunk(ci, carry):
        pltpu.sync_copy(h_hbm.at[cols_v.at[ci]], msgs_v)

        def grp(g, c2):
            w16 = w_v[ci, pl.ds(g * L, L)]
            for i in range(L):
                e = g * L + i
                wspl = lax.gather(
                    w16, jnp.full((L, 1), i, jnp.int32),
                    lax.GatherDimensionNumbers(
                        offset_dims=(), collapsed_slice_dims=(0,),
                        start_index_map=(0,)),
                    slice_sizes=(1,),
                    mode=lax.GatherScatterMode.PROMISE_IN_BOUNDS)
                for d in range(D // L):
                    sl = pl.ds(d * L, L)
                    msgs_v[e, sl] = msgs_v[e, sl] * wspl
            return c2

        lax.fori_loop(0, C // L, grp, 0)

        pltpu.sync_copy(msgs_v, acc.at[rows_v.at[ci]], add=True)
        return carry

    lax.fori_loop(0, cpw, chunk, 0)

    plsc.subcore_barrier()
    wbase = pl.multiple_of(sid * rpt, 8)
    pltpu.sync_copy(acc.at[pl.ds(wbase, rpt)],
                    p_hbm.at[cid, pl.ds(wbase, rpt)])
    if rem:
        @pl.when(sid == NS - 1)
        def _():
            pltpu.sync_copy(acc.at[pl.ds(NS * rpt, rem)],
                            p_hbm.at[cid, pl.ds(NS * rpt, rem)])


def kernel(x, edge_index, edge_weight, W, b):
    n, d = x.shape
    e = edge_weight.shape[0]
    C = 128

    blk = 1000 if n % 1000 == 0 else n
    h = pl.pallas_call(
        _matmul_body,
        grid=(n // blk,),
        in_specs=[
            pl.BlockSpec((blk, d), lambda i: (i, 0)),
            pl.BlockSpec((d, d), lambda i: (0, 0)),
            pl.BlockSpec((1, d), lambda i: (0, 0)),
        ],
        out_specs=pl.BlockSpec((blk, d), lambda i: (i, 0)),
        out_shape=jax.ShapeDtypeStruct((n, d), jnp.float32),
    )(x, W.T, b.reshape(1, d))

    per_w = -(-e // NW)
    cpw = -(-per_w // C)
    e_pad = NW * cpw * C
    pad = e_pad - e
    rows = jnp.concatenate([edge_index[0], jnp.zeros((pad,), jnp.int32)])
    cols = jnp.concatenate([edge_index[1], jnp.zeros((pad,), jnp.int32)])
    wgt = jnp.concatenate([edge_weight, jnp.zeros((pad,), jnp.float32)])
    rows3 = rows.reshape(NW, cpw, C)
    cols3 = cols.reshape(NW, cpw, C)
    wgt3 = wgt.reshape(NW, cpw, C)

    mesh = plsc.VectorSubcoreMesh(core_axis_name="c", subcore_axis_name="s")
    partials = pl.kernel(
        functools.partial(_edge_body, n, cpw),
        mesh=mesh,
        out_type=jax.ShapeDtypeStruct((NC, n, d), jnp.float32),
        scratch_types=[
            pltpu.VMEM((cpw, C), jnp.int32),
            pltpu.VMEM((cpw, C), jnp.int32),
            pltpu.VMEM((cpw, C), jnp.float32),
            pltpu.VMEM((C, d), jnp.float32),
            pltpu.VMEM_SHARED((n, d), jnp.float32),
        ],
    )(h, cols3, rows3, wgt3)

    out = pl.pallas_call(
        _fuse_body,
        grid=(n // blk,),
        in_specs=[
            pl.BlockSpec((blk, d), lambda i: (i, 0)),
            pl.BlockSpec((blk, d), lambda i: (i, 0)),
            pl.BlockSpec((blk, d), lambda i: (i, 0)),
        ],
        out_specs=pl.BlockSpec((blk, d), lambda i: (i, 0)),
        out_shape=jax.ShapeDtypeStruct((n, d), jnp.float32),
    )(partials[0], partials[1], x)
    return out
